# baseline (speedup 1.0000x reference)
import functools

import jax
import jax.numpy as jnp
from jax import lax
from jax.experimental import pallas as pl
from jax.experimental.pallas import tpu as pltpu
from jax.experimental.pallas import tpu_sc as plsc

_L = 128
_N = 32768
_K = 50
_NROWS = 384
_NTILES = 32
_RPT = _NROWS // _NTILES
_HBINS = 8192
_HSHIFT = 18
_CAP = 256
_FCAP = 272
_PAD = 64
_SELPAD = 80


def _sc_select_kernel(b_hbm, d_hbm, outb_hbm, outd_hbm,
                      bv, dv, hist, vm, flags, candv, candi, selidx,
                      selb, seld):
    wid = lax.axis_index("s") * 2 + lax.axis_index("c")
    iota = lax.iota(jnp.int32, 16)
    ones16 = jnp.ones((16,), jnp.int32)
    zeros16 = jnp.zeros((16,), jnp.int32)
    lane0 = iota == 0

    def row_body(j, _):
        row = wid * _RPT + j
        pltpu.sync_copy(b_hbm.at[row], bv)
        pltpu.sync_copy(d_hbm.at[row], dv)

        def zh(g, _c):
            for u in range(8):
                hist[pl.ds((g * 8 + u) * 16, 16)] = zeros16
            return 0
        lax.fori_loop(0, _HBINS // (8 * 16), zh, 0)

        def p1(g, rmax):
            for u in range(8):
                vecid = g * 8 + u
                base = vecid * 16
                b16 = bv[pl.ds(base, 16)]
                d16 = dv[pl.ds(base, 16)]
                p = jnp.abs(d16 - b16)
                pb = plsc.bitcast(p, jnp.int32)
                rmax = jnp.maximum(rmax, pb)
                vmx = jnp.max(pb) + zeros16
                plsc.store_compressed(vm.at[pl.ds(vecid, 16)], vmx,
                                      mask=lane0)
            return rmax
        rmax = lax.fori_loop(0, (_N // 16) // 8, p1, zeros16)
        mbits = jnp.max(rmax)
        mb = lax.shift_right_arithmetic(mbits, _HSHIFT)

        def vh(g, _c):
            for u in range(4):
                vv = vm[pl.ds((g * 4 + u) * 16, 16)]
                binv = lax.shift_right_arithmetic(vv, _HSHIFT)
                plsc.addupdate_scatter(hist, [binv], ones16)
            return 0
        lax.fori_loop(0, (_N // 256) // 4, vh, 0)

        def walk(w, carry):
            cum, cross_base, cross_cum = carry
            hi_b = mb - 16 * w
            cbase = jnp.maximum(hi_b - 15, 0)
            v = hist[pl.ds(cbase, 16)]
            bins = cbase + iota
            m = (bins <= hi_b) & (bins >= hi_b - 15)
            tot = jnp.sum(jnp.where(m, v, 0))
            newcum = cum + tot
            cond = (newcum >= _K) & (cum < _K)
            cross_base = jnp.where(cond, cbase, cross_base)
            cross_cum = jnp.where(cond, cum, cross_cum)
            return newcum, cross_base, cross_cum
        _, cross_base, cross_cum = lax.fori_loop(
            0, 64, walk, (jnp.int32(0), jnp.int32(0), jnp.int32(0)))

        cvec = hist[pl.ds(cross_base, 16)]
        cum = jnp.int32(0)
        bbin = jnp.int32(-1)
        for i in range(16):
            lane = 15 - i
            cum = cum + cvec[lane]
            cond = (cross_cum + cum >= _K) & (bbin < 0)
            bbin = jnp.where(cond, cross_base + lane, bbin)
        tau = lax.shift_left(jnp.maximum(bbin, 0), _HSHIFT)

        def fs(g, nf):
            for u in range(4):
                base = (g * 4 + u) * 16
                vv = vm[pl.ds(base, 16)]
                m = vv >= tau
                nfc = jnp.minimum(nf, _FCAP - 16)
                plsc.store_compressed(flags.at[pl.ds(nfc, 16)], base + iota,
                                      mask=m)
                nf = nf + plsc.all_reduce_population_count(m)[0]
            return nf
        nf = lax.fori_loop(0, (_N // 256) // 4, fs, jnp.int32(0))
        nf = jnp.minimum(nf, _FCAP - 16)

        for h in range(_CAP // 16):
            candv[pl.ds(h * 16, 16)] = zeros16 - 1
            candi[pl.ds(h * 16, 16)] = zeros16

        def collect(i, off):
            fvec = flags[pl.ds(i, 16)]
            vid = jnp.clip(fvec[0], 0, (_N // 16) - 1)
            base = vid * 16
            b16 = bv[pl.ds(base, 16)]
            d16 = dv[pl.ds(base, 16)]
            p = jnp.abs(d16 - b16)
            pb = plsc.bitcast(p, jnp.int32)
            mvec = pb >= tau
            offc = jnp.minimum(off, _CAP - 16)
            plsc.store_compressed(candv.at[pl.ds(offc, 16)], pb, mask=mvec)
            plsc.store_compressed(candi.at[pl.ds(offc, 16)], base + iota,
                                  mask=mvec)
            return off + plsc.all_reduce_population_count(mvec)[0]
        lax.fori_loop(0, nf, collect, jnp.int32(0))

        def tb(i, carry):
            lo, hi = carry
            mid = lo + lax.shift_right_logical(hi - lo + 1, 1)
            acc = zeros16
            for h in range(_CAP // 16):
                cv = candv[pl.ds(h * 16, 16)]
                acc = acc + (cv >= mid).astype(jnp.int32)
            pred = jnp.sum(acc) >= _K
            lo = jnp.where(pred, mid, lo)
            hi = jnp.where(pred, hi, mid - 1)
            return lo, hi
        tbits, _ = lax.fori_loop(
            0, 28, tb, (jnp.full((16,), tau, jnp.int32),
                        jnp.full((16,), mbits, jnp.int32)))

        accg = zeros16
        for h in range(_CAP // 16):
            cv = candv[pl.ds(h * 16, 16)]
            accg = accg + (cv > tbits).astype(jnp.int32)
        need = _K - jnp.sum(accg)

        def ib(i, carry):
            lo2, hi2 = carry
            mid2 = lax.shift_right_arithmetic(lo2 + hi2, 1)
            acc = zeros16
            for h in range(_CAP // 16):
                cv = candv[pl.ds(h * 16, 16)]
                ci = candi[pl.ds(h * 16, 16)]
                acc = acc + ((cv == tbits) & (ci <= mid2)).astype(jnp.int32)
            pred = jnp.sum(acc) >= need
            lo2 = jnp.where(pred, lo2, mid2 + 1)
            hi2 = jnp.where(pred, mid2, hi2)
            return lo2, hi2
        idx_cut, _ = lax.fori_loop(
            0, 15, ib, (zeros16, jnp.full((16,), _N - 1, jnp.int32)))

        for h in range(_SELPAD // 16):
            selidx[pl.ds(h * 16, 16)] = zeros16
        off2 = jnp.int32(0)
        for h in range(_CAP // 16):
            cv = candv[pl.ds(h * 16, 16)]
            ci = candi[pl.ds(h * 16, 16)]
            selm = (cv > tbits) | ((cv == tbits) & (ci <= idx_cut))
            offc2 = jnp.minimum(off2, _SELPAD - 16)
            plsc.store_compressed(selidx.at[pl.ds(offc2, 16)], ci, mask=selm)
            off2 = off2 + plsc.all_reduce_population_count(selm)[0]

        for h in range(_PAD // 16):
            iv = selidx[pl.ds(h * 16, 16)]
            gb = plsc.load_gather(bv, [iv])
            gd = plsc.load_gather(dv, [iv])
            m50 = (h * 16 + iota) < _K
            selb[pl.ds(h * 16, 16)] = jnp.where(m50, gb, 0.0)
            seld[pl.ds(h * 16, 16)] = jnp.where(m50, gd, 0.0)
        pltpu.sync_copy(selb, outb_hbm.at[row])
        pltpu.sync_copy(seld, outd_hbm.at[row])
        return 0

    lax.fori_loop(0, _RPT, row_body, 0)


def _sc_select(births, deaths):
    mesh = plsc.VectorSubcoreMesh(core_axis_name="c", subcore_axis_name="s")
    f = functools.partial(
        pl.kernel, mesh=mesh,
        compiler_params=pltpu.CompilerParams(needs_layout_passes=False),
        out_type=[jax.ShapeDtypeStruct((_NROWS, _PAD), jnp.float32),
                  jax.ShapeDtypeStruct((_NROWS, _PAD), jnp.float32)],
        scratch_types=[
            pltpu.VMEM((_N,), jnp.float32),
            pltpu.VMEM((_N,), jnp.float32),
            pltpu.VMEM((_HBINS,), jnp.int32),
            pltpu.VMEM((_N // 16 + 16,), jnp.int32),
            pltpu.VMEM((_FCAP,), jnp.int32),
            pltpu.VMEM((_CAP,), jnp.int32),
            pltpu.VMEM((_CAP,), jnp.int32),
            pltpu.VMEM((_SELPAD,), jnp.int32),
            pltpu.VMEM((_PAD,), jnp.float32),
            pltpu.VMEM((_PAD,), jnp.float32),
        ],
    )(_sc_select_kernel)
    return f(births, deaths)


def _final_kernel(b_ref, d_ref, w_ref, bias_ref, o_ref):
    cols = []
    for s in range(3):
        b = b_ref[:, s * _PAD:(s + 1) * _PAD]
        d = d_ref[:, s * _PAD:(s + 1) * _PAD]
        p = jnp.abs(d - b)
        logp = jnp.log1p(p)
        f0 = jnp.sum(p, axis=1)
        f1 = jnp.sum(b * p, axis=1)
        f2 = jnp.sum(d * p, axis=1)
        f3 = jnp.sum(b * logp, axis=1)
        f4 = jnp.sum(d * logp, axis=1)
        s5 = jnp.sum(jnp.where(p > 0.0, jnp.exp(p - 1.0), 0.0), axis=1)
        f5 = jnp.log(jnp.exp(jnp.float32(-1.0)) + s5) + 1.0
        cols.append(jnp.stack([f0, f1, f2, f3, f4, f5], axis=1))
    feats = jnp.concatenate(cols, axis=1)
    mean = jnp.mean(feats, axis=0, keepdims=True)
    var = jnp.mean((feats - mean) ** 2, axis=0, keepdims=True)
    normed = (feats - mean) / jnp.sqrt(var + 1e-5)
    out = jnp.sum(normed * w_ref[...], axis=1, keepdims=True) + bias_ref[0, 0]
    o_ref[...] = out


@jax.jit
def _run(births, deaths, W, bias):
    outb, outd = _sc_select(births, deaths)
    outb = outb.reshape(_L, 3 * _PAD)
    outd = outd.reshape(_L, 3 * _PAD)
    out = pl.pallas_call(
        _final_kernel,
        in_specs=[
            pl.BlockSpec((_L, 3 * _PAD), lambda: (0, 0)),
            pl.BlockSpec((_L, 3 * _PAD), lambda: (0, 0)),
            pl.BlockSpec((1, 18), lambda: (0, 0)),
            pl.BlockSpec((1, 1), lambda: (0, 0)),
        ],
        out_specs=pl.BlockSpec((_L, 1), lambda: (0, 0)),
        out_shape=jax.ShapeDtypeStruct((_L, 1), jnp.float32),
    )(outb, outd, W.reshape(1, 18), bias.reshape(1, 1))
    return out


def kernel(births, deaths, W, bias, k):
    return _run(births, deaths, W, bias)


# speedup vs baseline: 3.5586x; 3.5586x over previous
import functools

import jax
import jax.numpy as jnp
from jax import lax
from jax.experimental import pallas as pl
from jax.experimental.pallas import tpu as pltpu
from jax.experimental.pallas import tpu_sc as plsc

_L = 128
_N = 32768
_K = 50
_NROWS = 384
_NTILES = 32
_RPT = _NROWS // _NTILES
_HBINS = 8192
_HSHIFT = 18
_CAP = 128
_FCAP = 272
_PAD = 64
_SELPAD = 80


def _sc_select_kernel(b_hbm, d_hbm, outb_hbm, outd_hbm,
                      bv, dv, hist, vm, flags, candv, candi, selidx,
                      selb, seld):
    wid = lax.axis_index("s") * 2 + lax.axis_index("c")
    iota = lax.iota(jnp.int32, 16)
    ones16 = jnp.ones((16,), jnp.int32)
    zeros16 = jnp.zeros((16,), jnp.int32)

    def row_body(j, _):
        row = wid * _RPT + j
        pltpu.sync_copy(b_hbm.at[row], bv)
        pltpu.sync_copy(d_hbm.at[row], dv)

        def zh(g, _c):
            for u in range(8):
                hist[pl.ds((g * 8 + u) * 16, 16)] = zeros16
            return 0
        lax.fori_loop(0, _HBINS // (8 * 16), zh, 0)

        def p1(g, rmax):
            gm = zeros16
            for u in range(16):
                base = (g * 16 + u) * 16
                b16 = bv[pl.ds(base, 16)]
                d16 = dv[pl.ds(base, 16)]
                p = jnp.abs(d16 - b16)
                pb = plsc.bitcast(p, jnp.int32)
                gm = jnp.maximum(gm, pb)
            vm[pl.ds(g * 16, 16)] = gm
            return jnp.maximum(rmax, gm)
        rmax = lax.fori_loop(0, _N // 256, p1, zeros16)
        mbits = jnp.max(rmax)
        mb = lax.shift_right_arithmetic(mbits, _HSHIFT)

        def vh(g, _c):
            for u in range(4):
                vv = vm[pl.ds((g * 4 + u) * 16, 16)]
                binv = lax.shift_right_arithmetic(vv, _HSHIFT)
                plsc.addupdate_scatter(hist, [binv], ones16)
            return 0
        lax.fori_loop(0, (_N // 256) // 4, vh, 0)

        def walk(w, carry):
            cum, cross_base, cross_cum = carry
            hi_b = mb - 16 * w
            cbase = jnp.maximum(hi_b - 15, 0)
            v = hist[pl.ds(cbase, 16)]
            bins = cbase + iota
            m = (bins <= hi_b) & (bins >= hi_b - 15)
            tot = jnp.sum(jnp.where(m, v, 0))
            newcum = cum + tot
            cond = (newcum >= _K) & (cum < _K)
            cross_base = jnp.where(cond, cbase, cross_base)
            cross_cum = jnp.where(cond, cum, cross_cum)
            return newcum, cross_base, cross_cum
        _, cross_base, cross_cum = lax.fori_loop(
            0, 24, walk, (jnp.int32(0), jnp.int32(0), jnp.int32(0)))

        cvec = hist[pl.ds(cross_base, 16)]
        cum = jnp.int32(0)
        bbin = jnp.int32(-1)
        for i in range(16):
            lane = 15 - i
            cum = cum + cvec[lane]
            cond = (cross_cum + cum >= _K) & (bbin < 0)
            bbin = jnp.where(cond, cross_base + lane, bbin)
        tau = lax.shift_left(jnp.maximum(bbin, 0), _HSHIFT)

        def fs(g, nf):
            for u in range(4):
                base = (g * 4 + u) * 16
                vv = vm[pl.ds(base, 16)]
                m = vv >= tau
                nfc = jnp.minimum(nf, _FCAP - 16)
                plsc.store_compressed(flags.at[pl.ds(nfc, 16)], base + iota,
                                      mask=m)
                nf = nf + plsc.all_reduce_population_count(m)[0]
            return nf
        nf = lax.fori_loop(0, (_N // 256) // 4, fs, jnp.int32(0))
        nf = jnp.minimum(nf, _FCAP - 16)

        for h in range(_CAP // 16):
            candv[pl.ds(h * 16, 16)] = zeros16 - 1
            candi[pl.ds(h * 16, 16)] = zeros16

        def collect(i, off):
            fvec = flags[pl.ds(i, 16)]
            c = jnp.clip(fvec[0], 0, (_N // 16) - 1)
            idx16 = (lax.shift_left(lax.shift_right_arithmetic(c, 4), 8)
                     + lax.shift_left(iota, 4) + (c & 15))
            gb = plsc.load_gather(bv, [idx16])
            gd = plsc.load_gather(dv, [idx16])
            p = jnp.abs(gd - gb)
            pb = plsc.bitcast(p, jnp.int32)
            mvec = pb >= tau
            offc = jnp.minimum(off, _CAP - 16)
            plsc.store_compressed(candv.at[pl.ds(offc, 16)], pb, mask=mvec)
            plsc.store_compressed(candi.at[pl.ds(offc, 16)], idx16,
                                  mask=mvec)
            return off + plsc.all_reduce_population_count(mvec)[0]
        lax.fori_loop(0, nf, collect, jnp.int32(0))

        def tb(i, carry):
            lo, hi = carry
            mid = lo + lax.shift_right_logical(hi - lo + 1, 1)
            acc = zeros16
            for h in range(_CAP // 16):
                cv = candv[pl.ds(h * 16, 16)]
                acc = acc + (cv >= mid).astype(jnp.int32)
            pred = jnp.sum(acc) >= _K
            lo = jnp.where(pred, mid, lo)
            hi = jnp.where(pred, hi, mid - 1)
            return lo, hi
        tbits, _ = lax.fori_loop(
            0, 28, tb, (jnp.full((16,), tau, jnp.int32),
                        jnp.full((16,), mbits, jnp.int32)))

        accg = zeros16
        for h in range(_CAP // 16):
            cv = candv[pl.ds(h * 16, 16)]
            accg = accg + (cv > tbits).astype(jnp.int32)
        need = _K - jnp.sum(accg)

        def ib(i, carry):
            lo2, hi2 = carry
            mid2 = lax.shift_right_arithmetic(lo2 + hi2, 1)
            acc = zeros16
            for h in range(_CAP // 16):
                cv = candv[pl.ds(h * 16, 16)]
                ci = candi[pl.ds(h * 16, 16)]
                acc = acc + ((cv == tbits) & (ci <= mid2)).astype(jnp.int32)
            pred = jnp.sum(acc) >= need
            lo2 = jnp.where(pred, lo2, mid2 + 1)
            hi2 = jnp.where(pred, mid2, hi2)
            return lo2, hi2
        idx_cut, _ = lax.fori_loop(
            0, 15, ib, (zeros16, jnp.full((16,), _N - 1, jnp.int32)))

        for h in range(_SELPAD // 16):
            selidx[pl.ds(h * 16, 16)] = zeros16
        off2 = jnp.int32(0)
        for h in range(_CAP // 16):
            cv = candv[pl.ds(h * 16, 16)]
            ci = candi[pl.ds(h * 16, 16)]
            selm = (cv > tbits) | ((cv == tbits) & (ci <= idx_cut))
            offc2 = jnp.minimum(off2, _SELPAD - 16)
            plsc.store_compressed(selidx.at[pl.ds(offc2, 16)], ci, mask=selm)
            off2 = off2 + plsc.all_reduce_population_count(selm)[0]

        for h in range(_PAD // 16):
            iv = selidx[pl.ds(h * 16, 16)]
            gb = plsc.load_gather(bv, [iv])
            gd = plsc.load_gather(dv, [iv])
            m50 = (h * 16 + iota) < _K
            selb[pl.ds(h * 16, 16)] = jnp.where(m50, gb, 0.0)
            seld[pl.ds(h * 16, 16)] = jnp.where(m50, gd, 0.0)
        pltpu.sync_copy(selb, outb_hbm.at[row])
        pltpu.sync_copy(seld, outd_hbm.at[row])
        return 0

    lax.fori_loop(0, _RPT, row_body, 0)


def _sc_select(births, deaths):
    mesh = plsc.VectorSubcoreMesh(core_axis_name="c", subcore_axis_name="s")
    f = functools.partial(
        pl.kernel, mesh=mesh,
        compiler_params=pltpu.CompilerParams(needs_layout_passes=False),
        out_type=[jax.ShapeDtypeStruct((_NROWS, _PAD), jnp.float32),
                  jax.ShapeDtypeStruct((_NROWS, _PAD), jnp.float32)],
        scratch_types=[
            pltpu.VMEM((_N,), jnp.float32),
            pltpu.VMEM((_N,), jnp.float32),
            pltpu.VMEM((_HBINS,), jnp.int32),
            pltpu.VMEM((_N // 16 + 16,), jnp.int32),
            pltpu.VMEM((_FCAP,), jnp.int32),
            pltpu.VMEM((_CAP,), jnp.int32),
            pltpu.VMEM((_CAP,), jnp.int32),
            pltpu.VMEM((_SELPAD,), jnp.int32),
            pltpu.VMEM((_PAD,), jnp.float32),
            pltpu.VMEM((_PAD,), jnp.float32),
        ],
    )(_sc_select_kernel)
    return f(births, deaths)


def _final_kernel(b_ref, d_ref, w_ref, bias_ref, o_ref):
    cols = []
    for s in range(3):
        b = b_ref[:, s * _PAD:(s + 1) * _PAD]
        d = d_ref[:, s * _PAD:(s + 1) * _PAD]
        p = jnp.abs(d - b)
        logp = jnp.log1p(p)
        f0 = jnp.sum(p, axis=1)
        f1 = jnp.sum(b * p, axis=1)
        f2 = jnp.sum(d * p, axis=1)
        f3 = jnp.sum(b * logp, axis=1)
        f4 = jnp.sum(d * logp, axis=1)
        s5 = jnp.sum(jnp.where(p > 0.0, jnp.exp(p - 1.0), 0.0), axis=1)
        f5 = jnp.log(jnp.exp(jnp.float32(-1.0)) + s5) + 1.0
        cols.append(jnp.stack([f0, f1, f2, f3, f4, f5], axis=1))
    feats = jnp.concatenate(cols, axis=1)
    mean = jnp.mean(feats, axis=0, keepdims=True)
    var = jnp.mean((feats - mean) ** 2, axis=0, keepdims=True)
    normed = (feats - mean) / jnp.sqrt(var + 1e-5)
    out = jnp.sum(normed * w_ref[...], axis=1, keepdims=True) + bias_ref[0, 0]
    o_ref[...] = out


@jax.jit
def _run(births, deaths, W, bias):
    outb, outd = _sc_select(births, deaths)
    outb = outb.reshape(_L, 3 * _PAD)
    outd = outd.reshape(_L, 3 * _PAD)
    out = pl.pallas_call(
        _final_kernel,
        in_specs=[
            pl.BlockSpec((_L, 3 * _PAD), lambda: (0, 0)),
            pl.BlockSpec((_L, 3 * _PAD), lambda: (0, 0)),
            pl.BlockSpec((1, 18), lambda: (0, 0)),
            pl.BlockSpec((1, 1), lambda: (0, 0)),
        ],
        out_specs=pl.BlockSpec((_L, 1), lambda: (0, 0)),
        out_shape=jax.ShapeDtypeStruct((_L, 1), jnp.float32),
    )(outb, outd, W.reshape(1, 18), bias.reshape(1, 1))
    return out


def kernel(births, deaths, W, bias, k):
    return _run(births, deaths, W, bias)


# speedup vs baseline: 3.9264x; 1.1033x over previous
import functools

import jax
import jax.numpy as jnp
from jax import lax
from jax.experimental import pallas as pl
from jax.experimental.pallas import tpu as pltpu
from jax.experimental.pallas import tpu_sc as plsc

_L = 128
_N = 32768
_K = 50
_NROWS = 384
_NTILES = 32
_RPT = _NROWS // _NTILES
_HBINS = 8192
_HSHIFT = 18
_CAP = 128
_FCAP = 272
_PAD = 80


def _sc_select_kernel(b_hbm, d_hbm, outb_hbm, outd_hbm,
                      bv, dv, hist, vm, flags, candv, candi, candb, candd,
                      selb, seld, semb, semd):
    wid = lax.axis_index("s") * 2 + lax.axis_index("c")
    iota = lax.iota(jnp.int32, 16)
    ones16 = jnp.ones((16,), jnp.int32)
    zeros16 = jnp.zeros((16,), jnp.int32)

    row0 = wid * _RPT
    pltpu.async_copy(b_hbm.at[row0], bv, semb)
    pltpu.async_copy(d_hbm.at[row0], dv, semd)

    def row_body(j, _):
        row = wid * _RPT + j
        pltpu.make_async_copy(b_hbm.at[row], bv, semb).wait()
        pltpu.make_async_copy(d_hbm.at[row], dv, semd).wait()

        def zh(g, _c):
            for u in range(8):
                hist[pl.ds((g * 8 + u) * 16, 16)] = zeros16
            return 0
        lax.fori_loop(0, _HBINS // (8 * 16), zh, 0)

        def p1(g, rmax):
            gm = zeros16
            for u in range(16):
                base = (g * 16 + u) * 16
                b16 = bv[pl.ds(base, 16)]
                d16 = dv[pl.ds(base, 16)]
                p = jnp.abs(d16 - b16)
                pb = plsc.bitcast(p, jnp.int32)
                gm = jnp.maximum(gm, pb)
            vm[pl.ds(g * 16, 16)] = gm
            return jnp.maximum(rmax, gm)
        rmax = lax.fori_loop(0, _N // 256, p1, zeros16)
        mbits = jnp.max(rmax)
        mb = lax.shift_right_arithmetic(mbits, _HSHIFT)

        def vh(g, _c):
            for u in range(4):
                vv = vm[pl.ds((g * 4 + u) * 16, 16)]
                binv = lax.shift_right_arithmetic(vv, _HSHIFT)
                plsc.addupdate_scatter(hist, [binv], ones16)
            return 0
        lax.fori_loop(0, (_N // 256) // 4, vh, 0)

        def walk(w, carry):
            cum, cross_base, cross_cum = carry
            hi_b = mb - 16 * w
            cbase = jnp.maximum(hi_b - 15, 0)
            v = hist[pl.ds(cbase, 16)]
            bins = cbase + iota
            m = (bins <= hi_b) & (bins >= hi_b - 15)
            tot = jnp.sum(jnp.where(m, v, 0))
            newcum = cum + tot
            cond = (newcum >= _K) & (cum < _K)
            cross_base = jnp.where(cond, cbase, cross_base)
            cross_cum = jnp.where(cond, cum, cross_cum)
            return newcum, cross_base, cross_cum
        _, cross_base, cross_cum = lax.fori_loop(
            0, 24, walk, (jnp.int32(0), jnp.int32(0), jnp.int32(0)))

        cvec = hist[pl.ds(cross_base, 16)]
        cum = jnp.int32(0)
        bbin = jnp.int32(-1)
        for i in range(16):
            lane = 15 - i
            cum = cum + cvec[lane]
            cond = (cross_cum + cum >= _K) & (bbin < 0)
            bbin = jnp.where(cond, cross_base + lane, bbin)
        tau = lax.shift_left(jnp.maximum(bbin, 0), _HSHIFT)

        def fs(g, nf):
            for u in range(4):
                base = (g * 4 + u) * 16
                vv = vm[pl.ds(base, 16)]
                m = vv >= tau
                nfc = jnp.minimum(nf, _FCAP - 16)
                plsc.store_compressed(flags.at[pl.ds(nfc, 16)], base + iota,
                                      mask=m)
                nf = nf + plsc.all_reduce_population_count(m)[0]
            return nf
        nf = lax.fori_loop(0, (_N // 256) // 4, fs, jnp.int32(0))
        nf = jnp.minimum(nf, _FCAP - 16)

        for h in range(_CAP // 16):
            candv[pl.ds(h * 16, 16)] = zeros16 - 1
            candi[pl.ds(h * 16, 16)] = zeros16

        def collect(i, off):
            fvec = flags[pl.ds(i, 16)]
            c = jnp.clip(fvec[0], 0, (_N // 16) - 1)
            idx16 = (lax.shift_left(lax.shift_right_arithmetic(c, 4), 8)
                     + lax.shift_left(iota, 4) + (c & 15))
            gb = plsc.load_gather(bv, [idx16])
            gd = plsc.load_gather(dv, [idx16])
            p = jnp.abs(gd - gb)
            pb = plsc.bitcast(p, jnp.int32)
            mvec = pb >= tau
            offc = jnp.minimum(off, _CAP - 16)
            plsc.store_compressed(candv.at[pl.ds(offc, 16)], pb, mask=mvec)
            plsc.store_compressed(candi.at[pl.ds(offc, 16)], idx16,
                                  mask=mvec)
            plsc.store_compressed(candb.at[pl.ds(offc, 16)], gb, mask=mvec)
            plsc.store_compressed(candd.at[pl.ds(offc, 16)], gd, mask=mvec)
            return off + plsc.all_reduce_population_count(mvec)[0]
        lax.fori_loop(0, nf, collect, jnp.int32(0))

        @pl.when(j < _RPT - 1)
        def _prefetch():
            pltpu.async_copy(b_hbm.at[row + 1], bv, semb)
            pltpu.async_copy(d_hbm.at[row + 1], dv, semd)

        def tb(i, carry):
            lo, hi = carry
            mid = lo + lax.shift_right_logical(hi - lo + 1, 1)
            acc = zeros16
            for h in range(_CAP // 16):
                cv = candv[pl.ds(h * 16, 16)]
                acc = acc + (cv >= mid).astype(jnp.int32)
            pred = jnp.sum(acc) >= _K
            lo = jnp.where(pred, mid, lo)
            hi = jnp.where(pred, hi, mid - 1)
            return lo, hi
        tbits, _ = lax.fori_loop(
            0, 28, tb, (jnp.full((16,), tau, jnp.int32),
                        jnp.full((16,), mbits, jnp.int32)))

        accg = zeros16
        for h in range(_CAP // 16):
            cv = candv[pl.ds(h * 16, 16)]
            accg = accg + (cv > tbits).astype(jnp.int32)
        need = _K - jnp.sum(accg)

        def ib(i, carry):
            lo2, hi2 = carry
            mid2 = lax.shift_right_arithmetic(lo2 + hi2, 1)
            acc = zeros16
            for h in range(_CAP // 16):
                cv = candv[pl.ds(h * 16, 16)]
                ci = candi[pl.ds(h * 16, 16)]
                acc = acc + ((cv == tbits) & (ci <= mid2)).astype(jnp.int32)
            pred = jnp.sum(acc) >= need
            lo2 = jnp.where(pred, lo2, mid2 + 1)
            hi2 = jnp.where(pred, mid2, hi2)
            return lo2, hi2
        idx_cut, _ = lax.fori_loop(
            0, 15, ib, (zeros16, jnp.full((16,), _N - 1, jnp.int32)))

        for h in range(_PAD // 16):
            selb[pl.ds(h * 16, 16)] = jnp.zeros((16,), jnp.float32)
            seld[pl.ds(h * 16, 16)] = jnp.zeros((16,), jnp.float32)
        off2 = jnp.int32(0)
        for h in range(_CAP // 16):
            cv = candv[pl.ds(h * 16, 16)]
            ci = candi[pl.ds(h * 16, 16)]
            cb = candb[pl.ds(h * 16, 16)]
            cd = candd[pl.ds(h * 16, 16)]
            selm = (cv > tbits) | ((cv == tbits) & (ci <= idx_cut))
            offc2 = jnp.minimum(off2, _PAD - 16)
            plsc.store_compressed(selb.at[pl.ds(offc2, 16)], cb, mask=selm)
            plsc.store_compressed(seld.at[pl.ds(offc2, 16)], cd, mask=selm)
            off2 = off2 + plsc.all_reduce_population_count(selm)[0]
        pltpu.sync_copy(selb, outb_hbm.at[row])
        pltpu.sync_copy(seld, outd_hbm.at[row])
        return 0

    lax.fori_loop(0, _RPT, row_body, 0)


def _sc_select(births, deaths):
    mesh = plsc.VectorSubcoreMesh(core_axis_name="c", subcore_axis_name="s")
    f = functools.partial(
        pl.kernel, mesh=mesh,
        compiler_params=pltpu.CompilerParams(needs_layout_passes=False),
        out_type=[jax.ShapeDtypeStruct((_NROWS, _PAD), jnp.float32),
                  jax.ShapeDtypeStruct((_NROWS, _PAD), jnp.float32)],
        scratch_types=[
            pltpu.VMEM((_N,), jnp.float32),
            pltpu.VMEM((_N,), jnp.float32),
            pltpu.VMEM((_HBINS,), jnp.int32),
            pltpu.VMEM((_N // 16 + 16,), jnp.int32),
            pltpu.VMEM((_FCAP,), jnp.int32),
            pltpu.VMEM((_CAP,), jnp.int32),
            pltpu.VMEM((_CAP,), jnp.int32),
            pltpu.VMEM((_CAP,), jnp.float32),
            pltpu.VMEM((_CAP,), jnp.float32),
            pltpu.VMEM((_PAD,), jnp.float32),
            pltpu.VMEM((_PAD,), jnp.float32),
            pltpu.SemaphoreType.DMA,
            pltpu.SemaphoreType.DMA,
        ],
    )(_sc_select_kernel)
    return f(births, deaths)


def _final_kernel(b_ref, d_ref, w_ref, bias_ref, o_ref):
    cols = []
    for s in range(3):
        b = b_ref[:, s * _PAD:(s + 1) * _PAD]
        d = d_ref[:, s * _PAD:(s + 1) * _PAD]
        p = jnp.abs(d - b)
        logp = jnp.log1p(p)
        f0 = jnp.sum(p, axis=1)
        f1 = jnp.sum(b * p, axis=1)
        f2 = jnp.sum(d * p, axis=1)
        f3 = jnp.sum(b * logp, axis=1)
        f4 = jnp.sum(d * logp, axis=1)
        s5 = jnp.sum(jnp.where(p > 0.0, jnp.exp(p - 1.0), 0.0), axis=1)
        f5 = jnp.log(jnp.exp(jnp.float32(-1.0)) + s5) + 1.0
        cols.append(jnp.stack([f0, f1, f2, f3, f4, f5], axis=1))
    feats = jnp.concatenate(cols, axis=1)
    mean = jnp.mean(feats, axis=0, keepdims=True)
    var = jnp.mean((feats - mean) ** 2, axis=0, keepdims=True)
    normed = (feats - mean) / jnp.sqrt(var + 1e-5)
    out = jnp.sum(normed * w_ref[...], axis=1, keepdims=True) + bias_ref[0, 0]
    o_ref[...] = out


@jax.jit
def _run(births, deaths, W, bias):
    outb, outd = _sc_select(births, deaths)
    outb = outb.reshape(_L, 3 * _PAD)
    outd = outd.reshape(_L, 3 * _PAD)
    out = pl.pallas_call(
        _final_kernel,
        in_specs=[
            pl.BlockSpec((_L, 3 * _PAD), lambda: (0, 0)),
            pl.BlockSpec((_L, 3 * _PAD), lambda: (0, 0)),
            pl.BlockSpec((1, 18), lambda: (0, 0)),
            pl.BlockSpec((1, 1), lambda: (0, 0)),
        ],
        out_specs=pl.BlockSpec((_L, 1), lambda: (0, 0)),
        out_shape=jax.ShapeDtypeStruct((_L, 1), jnp.float32),
    )(outb, outd, W.reshape(1, 18), bias.reshape(1, 1))
    return out


def kernel(births, deaths, W, bias, k):
    return _run(births, deaths, W, bias)
